# Initial kernel scaffold; baseline (speedup 1.0000x reference)
import jax
import jax.numpy as jnp
from jax.experimental import pallas as pl


def kernel(x0, edge_index0, edge_attr0, batch0, x1, edge_index1, edge_attr1, batch1, atom_emb1, atom_emb2, edge_emb1, edge_emb2, W1, b1, W2, b2, bn_gamma, bn_beta, P1, pb1, P2, pb2):
    raise NotImplementedError("write your pallas kernel here")


# speedup vs baseline: 1.9839x; 1.9839x over previous
import functools
import numpy as np
import jax
import jax.numpy as jnp
from jax import lax
from jax.experimental import pallas as pl
from jax.experimental.pallas import tpu as pltpu, tpu_sc as plsc

N = 10000
E = 160000
NG = 256
EMB = 300
D = 384
DH = 640
NL = 5
EP = 172032
NDUM = EP - (E + N)
NPAD = 12288
CH = 128
WIN = 144
NCHUNK = EP // CH
NBLK = 400
NNB = N // NBLK
NA = 128

SC_NW = 32
GCH = 128
BPW = EP // SC_NW
GIT = BPW // GCH

_F32 = jnp.float32


def _sc_gather_body(table_hbm, idx_hbm, out_hbm, idx_v, rows_v, sem):
    wid = lax.axis_index("s") * 2 + lax.axis_index("c")
    base = wid * BPW

    def step(i, _):
        off = base + i * GCH
        pltpu.sync_copy(idx_hbm.at[pl.ds(off, GCH)], idx_v)
        pltpu.async_copy(table_hbm.at[idx_v], rows_v, sem).wait()
        pltpu.sync_copy(rows_v, out_hbm.at[pl.ds(off, GCH)])
        return _

    lax.fori_loop(0, GIT, step, None)


@functools.lru_cache(maxsize=1)
def _sc_gather_kernel():
    return functools.partial(
        pl.kernel,
        mesh=plsc.VectorSubcoreMesh(core_axis_name="c", subcore_axis_name="s"),
        out_type=jax.ShapeDtypeStruct((EP, D), _F32),
        scratch_types=[
            pltpu.VMEM((GCH,), jnp.int32),
            pltpu.VMEM((GCH, D), _F32),
            pltpu.SemaphoreType.DMA,
        ],
    )(_sc_gather_body)


def _sc_gather(table, idx):
    return _sc_gather_kernel()(table, idx)


def _h0_body(xa_ref, xb_ref, a1_ref, a2_ref, out_ref):
    ia = xa_ref[0, 0, :]
    ib = xb_ref[0, 0, :]
    col = lax.broadcasted_iota(jnp.int32, (NBLK, NA), 1)
    oh1 = (ia[:, None] == col).astype(_F32)
    oh2 = (ib[:, None] == col).astype(_F32)
    out_ref[...] = (jnp.dot(oh1, a1_ref[...], preferred_element_type=_F32)
                    + jnp.dot(oh2, a2_ref[...], preferred_element_type=_F32))


def _h0(xa3, xb3, a1p, a2p):
    return pl.pallas_call(
        _h0_body,
        grid=(NNB,),
        in_specs=[
            pl.BlockSpec((1, 1, NBLK), lambda i: (i, 0, 0)),
            pl.BlockSpec((1, 1, NBLK), lambda i: (i, 0, 0)),
            pl.BlockSpec((NA, D), lambda i: (0, 0)),
            pl.BlockSpec((NA, D), lambda i: (0, 0)),
        ],
        out_specs=pl.BlockSpec((NBLK, D), lambda i: (i, 0)),
        out_shape=jax.ShapeDtypeStruct((N, D), _F32),
    )(xa3, xb3, a1p, a2p)


def _segreduce_body(relu_in, bases_ref, g_ref, dst_ref, sc_ref, sh_ref, out_ref):
    pid = pl.program_id(0)

    @pl.when(pid == 0)
    def _():
        out_ref[...] = jnp.zeros((NPAD, D), _F32)

    base = pl.multiple_of((bases_ref[pid] // 8) * 8, 8)
    h = g_ref[...] * sc_ref[...] + sh_ref[...]
    if relu_in:
        h = jnp.maximum(h, 0.0)
    rel = dst_ref[0, 0, :] - base
    row = lax.broadcasted_iota(jnp.int32, (WIN, CH), 0)
    oht = (row == rel[None, :]).astype(_F32)
    seg = jnp.dot(oht, h, preferred_element_type=_F32)
    out_ref[pl.ds(base, WIN), :] += seg


def _segreduce(bases, gathered, dst3, scale, shift, relu_in):
    body = functools.partial(_segreduce_body, relu_in)
    return pl.pallas_call(
        body,
        grid_spec=pltpu.PrefetchScalarGridSpec(
            num_scalar_prefetch=1,
            grid=(NCHUNK,),
            in_specs=[
                pl.BlockSpec((CH, D), lambda i, b: (i, 0)),
                pl.BlockSpec((1, 1, CH), lambda i, b: (i, 0, 0)),
                pl.BlockSpec((1, D), lambda i, b: (0, 0)),
                pl.BlockSpec((1, D), lambda i, b: (0, 0)),
            ],
            out_specs=pl.BlockSpec((NPAD, D), lambda i, b: (0, 0)),
        ),
        out_shape=jax.ShapeDtypeStruct((NPAD, D), _F32),
    )(bases, gathered, dst3, scale, shift)


def _ccnt_body(bases_ref, dst_ref, c_ref, out_ref):
    pid = pl.program_id(0)

    @pl.when(pid == 0)
    def _():
        out_ref[...] = jnp.zeros((NPAD, NA), _F32)

    base = pl.multiple_of((bases_ref[pid] // 8) * 8, 8)
    rel = dst_ref[0, 0, :] - base
    row = lax.broadcasted_iota(jnp.int32, (WIN, CH), 0)
    oht = (row == rel[None, :]).astype(_F32)
    col = lax.broadcasted_iota(jnp.int32, (CH, NA), 1)
    ohc = (c_ref[0, 0, :][:, None] == col).astype(_F32)
    out_ref[pl.ds(base, WIN), :] += jnp.dot(oht, ohc, preferred_element_type=_F32)


def _ccnt(bases, dst3, c3):
    return pl.pallas_call(
        _ccnt_body,
        grid_spec=pltpu.PrefetchScalarGridSpec(
            num_scalar_prefetch=1,
            grid=(NCHUNK,),
            in_specs=[
                pl.BlockSpec((1, 1, CH), lambda i, b: (i, 0, 0)),
                pl.BlockSpec((1, 1, CH), lambda i, b: (i, 0, 0)),
            ],
            out_specs=pl.BlockSpec((NPAD, NA), lambda i, b: (0, 0)),
        ),
        out_shape=jax.ShapeDtypeStruct((NPAD, NA), _F32),
    )(bases, dst3, c3)


def _mlp_body(aggr_ref, cc_ref, ec_ref, w1_ref, b1_ref, w2_ref, b2_ref,
              h2_ref, st_ref):
    pid = pl.program_id(0)
    a = aggr_ref[...] + jnp.dot(cc_ref[...], ec_ref[...],
                                preferred_element_type=_F32)
    mid = jnp.maximum(jnp.dot(a, w1_ref[...], preferred_element_type=_F32)
                      + b1_ref[...], 0.0)
    h2 = jnp.dot(mid, w2_ref[...], preferred_element_type=_F32) + b2_ref[...]
    h2_ref[...] = h2

    @pl.when(pid == 0)
    def _():
        st_ref[...] = jnp.zeros((8, D), _F32)

    s1 = jnp.sum(h2, axis=0, keepdims=True)
    s2 = jnp.sum(h2 * h2, axis=0, keepdims=True)
    st_ref[...] += jnp.concatenate([s1, s2, jnp.zeros((6, D), _F32)], axis=0)


def _mlp(aggr, ccnt, ecl, w1l, b1l, w2l, b2l):
    return pl.pallas_call(
        _mlp_body,
        grid=(NNB,),
        in_specs=[
            pl.BlockSpec((NBLK, D), lambda i: (i, 0)),
            pl.BlockSpec((NBLK, NA), lambda i: (i, 0)),
            pl.BlockSpec((NA, D), lambda i: (0, 0)),
            pl.BlockSpec((D, DH), lambda i: (0, 0)),
            pl.BlockSpec((1, DH), lambda i: (0, 0)),
            pl.BlockSpec((DH, D), lambda i: (0, 0)),
            pl.BlockSpec((1, D), lambda i: (0, 0)),
        ],
        out_specs=[
            pl.BlockSpec((NBLK, D), lambda i: (i, 0)),
            pl.BlockSpec((8, D), lambda i: (0, 0)),
        ],
        out_shape=[
            jax.ShapeDtypeStruct((N, D), _F32),
            jax.ShapeDtypeStruct((8, D), _F32),
        ],
    )(aggr, ccnt, ecl, w1l, b1l, w2l, b2l)


def _pool_body(h_ref, b_ref, sc_ref, sh_ref, pool_ref, cnt_ref):
    pid = pl.program_id(0)

    @pl.when(pid == 0)
    def _():
        pool_ref[...] = jnp.zeros((NG, D), _F32)
        cnt_ref[...] = jnp.zeros((NG, 8), _F32)

    h = h_ref[...] * sc_ref[...] + sh_ref[...]
    bv = b_ref[0, 0, :]
    row = lax.broadcasted_iota(jnp.int32, (NG, NBLK), 0)
    oht = (row == bv[None, :]).astype(_F32)
    pool_ref[...] += jnp.dot(oht, h, preferred_element_type=_F32)
    cnt_ref[...] += jnp.dot(oht, jnp.ones((NBLK, 8), _F32),
                            preferred_element_type=_F32)


def _pool(h, batch3, scale, shift):
    return pl.pallas_call(
        _pool_body,
        grid=(NNB,),
        in_specs=[
            pl.BlockSpec((NBLK, D), lambda i: (i, 0)),
            pl.BlockSpec((1, 1, NBLK), lambda i: (i, 0, 0)),
            pl.BlockSpec((1, D), lambda i: (0, 0)),
            pl.BlockSpec((1, D), lambda i: (0, 0)),
        ],
        out_specs=[
            pl.BlockSpec((NG, D), lambda i: (0, 0)),
            pl.BlockSpec((NG, 8), lambda i: (0, 0)),
        ],
        out_shape=[
            jax.ShapeDtypeStruct((NG, D), _F32),
            jax.ShapeDtypeStruct((NG, 8), _F32),
        ],
    )(h, batch3, scale, shift)


def _proj_body(pool_ref, cnt_ref, p1_ref, pb1_ref, p2_ref, pb2_ref, out_ref):
    g = pool_ref[...] / jnp.maximum(cnt_ref[...][:, 0:1], 1.0)
    z = jnp.maximum(jnp.dot(g, p1_ref[...], preferred_element_type=_F32)
                    + pb1_ref[...], 0.0)
    z = jnp.dot(z, p2_ref[...], preferred_element_type=_F32) + pb2_ref[...]
    nrm = jnp.sqrt(jnp.sum(z * z, axis=1, keepdims=True))
    out_ref[...] = z / jnp.maximum(nrm, 1e-12)


def _proj(pooled, cnt, p1p, pb1p, p2p, pb2p):
    return pl.pallas_call(
        _proj_body,
        out_shape=jax.ShapeDtypeStruct((NG, D), _F32),
    )(pooled, cnt, p1p, pb1p, p2p, pb2p)


def _sim_body(f_ref, out_ref):
    f = f_ref[...]
    sim = lax.dot_general(f, f, (((1,), (1,)), ((), ())),
                          preferred_element_type=_F32)
    out_ref[...] = sim * 25.0


def _sim(f):
    return pl.pallas_call(
        _sim_body,
        out_shape=jax.ShapeDtypeStruct((2 * NG, 2 * NG), _F32),
    )(f)


def _edge_prep(edge_index, edge_attr):
    sl = jnp.arange(N, dtype=jnp.int32)
    src = jnp.concatenate([edge_index[0].astype(jnp.int32), sl])
    dst = jnp.concatenate([edge_index[1].astype(jnp.int32), sl])
    c = jnp.concatenate([
        edge_attr[:, 0].astype(jnp.int32) * 3 + edge_attr[:, 1].astype(jnp.int32),
        jnp.full((N,), 12, jnp.int32)])
    dum = jnp.arange(NDUM, dtype=jnp.int32) + N
    src = jnp.concatenate([src, jnp.zeros((NDUM,), jnp.int32)])
    dst = jnp.concatenate([dst, dum])
    c = jnp.concatenate([c, jnp.zeros((NDUM,), jnp.int32)])
    perm = jnp.argsort(dst)
    src_s = src[perm]
    dst_s = dst[perm]
    c_s = c[perm]
    bases = dst_s[::CH]
    return src_s, bases, dst_s.reshape(NCHUNK, 1, CH), c_s.reshape(NCHUNK, 1, CH)


def _padw(w, r, cshape):
    out = jnp.zeros(cshape, _F32)
    return out.at[tuple(slice(0, s) for s in w.shape)].set(w.astype(_F32)) if r else out


def _build_cols():
    n2 = 2 * NG
    cols = np.zeros((n2, n2 - 1), np.int32)
    for i in range(n2):
        p = (i + NG) % n2
        negs = [j for j in range(n2) if j != i and j != p]
        cols[i, 0] = p
        cols[i, 1:] = negs
    return jnp.asarray(cols)


_COLS = _build_cols()


def _features(x, edge_index, edge_attr, batch, wp):
    src_s, bases, dst3, c3 = _edge_prep(edge_index, edge_attr)
    xa3 = x[:, 0].astype(jnp.int32).reshape(NNB, 1, NBLK)
    xb3 = x[:, 1].astype(jnp.int32).reshape(NNB, 1, NBLK)
    batch3 = batch.astype(jnp.int32).reshape(NNB, 1, NBLK)

    h = _h0(xa3, xb3, wp['a1'], wp['a2'])
    ccnt = _ccnt(bases, dst3, c3)

    scale = jnp.ones((1, D), _F32)
    shift = jnp.zeros((1, D), _F32)
    for l in range(NL):
        gathered = _sc_gather(h, src_s)
        aggr = _segreduce(bases, gathered, dst3, scale, shift, relu_in=(l > 0))
        h, st = _mlp(aggr, ccnt, wp['ec'][l], wp['w1'][l], wp['b1'][l],
                     wp['w2'][l], wp['b2'][l])
        mean = st[0] / N
        var = jnp.maximum(st[1] / N - mean * mean, 0.0)
        scale = (wp['gam'][l] / jnp.sqrt(var + 1e-5))[None, :]
        shift = wp['bet'][l][None, :] - mean[None, :] * scale

    pooled, cnt = _pool(h, batch3, scale, shift)
    return _proj(pooled, cnt, wp['p1'], wp['pb1'], wp['p2'], wp['pb2'])


def kernel(x0, edge_index0, edge_attr0, batch0, x1, edge_index1, edge_attr1,
           batch1, atom_emb1, atom_emb2, edge_emb1, edge_emb2, W1, b1, W2, b2,
           bn_gamma, bn_beta, P1, pb1, P2, pb2):
    a_idx = np.arange(18) // 3
    b_idx = np.arange(18) % 3
    ecomb = edge_emb1[:, a_idx, :] + edge_emb2[:, b_idx, :]
    wp = dict(
        a1=_padw(atom_emb1, 1, (NA, D)),
        a2=_padw(atom_emb2, 1, (NA, D)),
        ec=_padw(ecomb, 1, (NL, NA, D)),
        w1=_padw(W1, 1, (NL, D, DH)),
        b1=_padw(b1[:, None, :], 1, (NL, 1, DH)),
        w2=_padw(W2, 1, (NL, DH, D)),
        b2=_padw(b2[:, None, :], 1, (NL, 1, D)),
        gam=_padw(bn_gamma, 1, (NL, D)),
        bet=_padw(bn_beta, 1, (NL, D)),
        p1=_padw(P1, 1, (D, D)),
        pb1=_padw(pb1[None, :], 1, (1, D)),
        p2=_padw(P2, 1, (D, D)),
        pb2=_padw(pb2[None, :], 1, (1, D)),
    )
    f0 = _features(x0, edge_index0, edge_attr0, batch0, wp)
    f1 = _features(x1, edge_index1, edge_attr1, batch1, wp)
    f = jnp.concatenate([f0, f1], axis=0)
    sim = _sim(f)
    logits = jnp.take_along_axis(sim, _COLS, axis=1)
    labels = jnp.zeros((2 * NG,), jnp.int32)
    return logits, labels
